# baseline (speedup 1.0000x reference)
import functools

import jax
import jax.numpy as jnp
from jax import lax
from jax.experimental import pallas as pl
from jax.experimental.pallas import tpu as pltpu
from jax.experimental.pallas import tpu_sc as plsc

B = 32
KX = 30
NTRI = 512
NW = 32
TPW = NTRI // NW
RPW = 3 * TPW
L = 16

_mesh = plsc.VectorSubcoreMesh(core_axis_name="c", subcore_axis_name="s")

D = 49152
BAND = D // 4


def _prep_body(ev_hbm, o_ref, s0, s1, s2, s3, sem):
    i = pl.program_id(0)
    cps = [pltpu.make_async_copy(
        ev_hbm.at[pl.ds((i + 12 * q) * 1024, 1024), :], sq, sem)
        for q, sq in enumerate((s0, s1, s2, s3))]
    for cp in cps:
        cp.start()
    for cp in cps:
        cp.wait()
    o_ref[...] = jnp.concatenate(
        [sq[:, :32] for sq in (s0, s1, s2, s3)], axis=1)


_prep = pl.pallas_call(
    _prep_body,
    grid=(12,),
    in_specs=[pl.BlockSpec(memory_space=pl.ANY)],
    out_specs=pl.BlockSpec((1024, 128), lambda i: (i, 0)),
    out_shape=jax.ShapeDtypeStruct((BAND, 128), jnp.float32),
    scratch_shapes=[pltpu.VMEM((1024, 64), jnp.float32)] * 4
    + [pltpu.SemaphoreType.DMA],
)


def _nsqrt(x):
    i = lax.bitcast_convert_type(x, jnp.int32)
    y = lax.bitcast_convert_type(
        lax.shift_right_arithmetic(i, 1) + jnp.int32(0x1FBD1DF5), jnp.float32)
    for _ in range(3):
        y = 0.5 * (y + x / y)
    return y


@functools.partial(
    pl.kernel,
    out_type=jax.ShapeDtypeStruct((2, B, L), jnp.float32),
    mesh=_mesh,
    compiler_params=pltpu.CompilerParams(needs_layout_passes=False,
                                         use_tc_tiling_on_sc=True),
    scratch_types=[
        pltpu.VMEM((RPW,), jnp.int32),
        pltpu.VMEM((RPW,), jnp.int32),
        pltpu.VMEM((RPW,), jnp.int32),
        pltpu.VMEM((RPW, 128), jnp.float32),
        pltpu.VMEM((RPW, 128), jnp.float32),
        pltpu.VMEM((8 + B * KX, ), jnp.float32),
        pltpu.VMEM((2 * B,), jnp.int32),
        pltpu.VMEM((2 * B, 128), jnp.float32),
        pltpu.VMEM((B, L), jnp.float32),
        pltpu.VMEM((16, B, L), jnp.float32),
        pltpu.VMEM((B, L), jnp.float32),
        pltpu.HBM((2, 16, B, L), jnp.float32),
        pltpu.SemaphoreType.DMA,
        pltpu.SemaphoreType.DMA,
        pltpu.SemaphoreType.DMA,
        pltpu.SemaphoreType.DMA,
    ],
)
def _sc_loss(ev2_hbm, rlt_hbm, x_hbm, tgt_hbm, mean_hbm, out_hbm,
             ridx_v, eidx_v, midx_v, g_v, mean_v, x_v, tidx_v, t_v, part_v,
             red_v, osum_v, stage, sem_g, sem_m, sem_t, sem_x):
    c = lax.axis_index("c")
    s = lax.axis_index("s")
    g = c * 16 + s
    base = g * RPW

    iota = lax.iota(jnp.int32, L)

    cp_r = pltpu.async_copy(rlt_hbm.at[pl.ds(base, RPW)], ridx_v, sem_m)
    cp_x = pltpu.async_copy(x_hbm, x_v.at[pl.ds(8, B * KX)], sem_x)
    trow = lax.shift_right_logical(base, 7)
    for ch in range(2 * B // L):
        n = iota + ch * L
        row = 12 * lax.shift_right_logical(n, 1) + trow + (n & 1)
        tidx_v[pl.ds(ch * L, L)] = jnp.minimum(row, 383)
    cp_t = pltpu.async_copy(tgt_hbm.at[tidx_v], t_v, sem_t)
    cp_r.wait()
    for ch in range(RPW // L):
        r = ridx_v[pl.ds(ch * L, L)]
        q = ((r >= BAND).astype(jnp.int32)
             + (r >= 2 * BAND).astype(jnp.int32)
             + (r >= 3 * BAND).astype(jnp.int32))
        eidx_v[pl.ds(ch * L, L)] = r - q * BAND
        midx_v[pl.ds(ch * L, L)] = lax.shift_right_logical(r, 7)
    cp_g = pltpu.async_copy(ev2_hbm.at[eidx_v], g_v, sem_g)
    cp_m = pltpu.async_copy(mean_hbm.at[midx_v], mean_v, sem_m)
    cp_x.wait()
    cp_g.wait()
    cp_m.wait()
    cp_t.wait()

    row3 = [iota * 3 + j for j in range(3)]
    rj = [plsc.load_gather(ridx_v, [row3[j]]) for j in range(3)]
    mt = [plsc.load_gather(mean_v, [row3[j], rj[j] & 127]) for j in range(3)]
    colb = [32 * ((rj[j] >= BAND).astype(jnp.int32)
                  + (rj[j] >= 2 * BAND).astype(jnp.int32)
                  + (rj[j] >= 3 * BAND).astype(jnp.int32)) for j in range(3)]
    toff = (base & 127) + iota * 3
    trow2 = lax.shift_right_logical(toff, 7)
    tcol = toff & 127

    for bc in range(B // 8):
        acc = [[mt[j] for _ in range(8)] for j in range(3)]
        for k in range(30):
            col = [plsc.load_gather(g_v, [row3[j], colb[j] + k])
                   for j in range(3)]
            for b8 in range(8):
                b = bc * 8 + b8
                xv = plsc.load_gather(
                    x_v, [jnp.full((L,), 8 + b * KX + k, jnp.int32)])
                for j in range(3):
                    acc[j][b8] = acc[j][b8] + col[j] * xv
        for b8 in range(8):
            b = bc * 8 + b8
            tt = plsc.load_gather(t_v, [trow2 + 2 * b, tcol])
            d0 = acc[0][b8] - tt
            d1 = acc[1][b8] - tt
            d2 = acc[2][b8] - tt
            dist = _nsqrt(d0 * d0 + d1 * d1 + d2 * d2)
            part_v[b, :] = jnp.broadcast_to(jnp.sum(dist), (L,))

    pltpu.sync_copy(part_v, stage.at[c, s])
    plsc.subcore_barrier()

    @pl.when(s == 0)
    def _fold():
        cps = [pltpu.async_copy(stage.at[c, i], red_v.at[i], sem_g)
               for i in range(16)]
        for cp in cps:
            cp.wait()
        for b in range(B):
            acc_r = red_v[0, b, :]
            for i in range(1, 16):
                acc_r = acc_r + red_v[i, b, :]
            osum_v[b, :] = acc_r
        pltpu.sync_copy(osum_v, out_hbm.at[c])


def kernel(output, target, eigenVectors, mean, indices, outline, rightLineIdxs):
    ev2 = _prep(eigenVectors)
    tgt2 = target.reshape(-1, 128)
    mean2 = mean.reshape(-1, 128)
    part = _sc_loss(ev2, rightLineIdxs, output.reshape(-1), tgt2, mean2)
    return (part[0] + part[1])[:, 0]


# speedup vs baseline: 1.4150x; 1.4150x over previous
import functools

import jax
import jax.numpy as jnp
from jax import lax
from jax.experimental import pallas as pl
from jax.experimental.pallas import tpu as pltpu
from jax.experimental.pallas import tpu_sc as plsc

B = 32
KX = 30
NTRI = 512
NW = 32
TPW = NTRI // NW
RPW = 3 * TPW
L = 16

_mesh = plsc.VectorSubcoreMesh(core_axis_name="c", subcore_axis_name="s")

D = 49152
BAND = D // 4


def _nsqrt(x):
    i = lax.bitcast_convert_type(x, jnp.int32)
    y = lax.bitcast_convert_type(
        lax.shift_right_arithmetic(i, 1) + jnp.int32(0x1FBD1DF5), jnp.float32)
    for _ in range(3):
        y = 0.5 * (y + x / y)
    return y


@functools.partial(
    pl.kernel,
    out_type=jax.ShapeDtypeStruct((2, B, L), jnp.float32),
    mesh=_mesh,
    compiler_params=pltpu.CompilerParams(needs_layout_passes=False,
                                         use_tc_tiling_on_sc=True),
    scratch_types=[
        pltpu.VMEM((RPW,), jnp.int32),
        pltpu.VMEM((RPW,), jnp.int32),
        pltpu.VMEM((RPW, 64), jnp.float32),
        pltpu.VMEM((RPW, 128), jnp.float32),
        pltpu.VMEM((8 + B * KX, ), jnp.float32),
        pltpu.VMEM((2 * B,), jnp.int32),
        pltpu.VMEM((2 * B, 128), jnp.float32),
        pltpu.VMEM((B, L), jnp.float32),
        pltpu.VMEM((16, B, L), jnp.float32),
        pltpu.VMEM((B, L), jnp.float32),
        pltpu.HBM((2, 16, B, L), jnp.float32),
        pltpu.SemaphoreType.DMA,
        pltpu.SemaphoreType.DMA,
        pltpu.SemaphoreType.DMA,
        pltpu.SemaphoreType.DMA,
    ],
)
def _sc_loss(ev2_hbm, rlt_hbm, x_hbm, tgt_hbm, mean_hbm, out_hbm,
             ridx_v, midx_v, g_v, mean_v, x_v, tidx_v, t_v, part_v,
             red_v, osum_v, stage, sem_g, sem_m, sem_t, sem_x):
    c = lax.axis_index("c")
    s = lax.axis_index("s")
    g = c * 16 + s
    base = g * RPW

    iota = lax.iota(jnp.int32, L)

    cp_r = pltpu.async_copy(rlt_hbm.at[pl.ds(base, RPW)], ridx_v, sem_m)
    cp_x = pltpu.async_copy(x_hbm, x_v.at[pl.ds(8, B * KX)], sem_x)
    trow = lax.shift_right_logical(base, 7)
    for ch in range(2 * B // L):
        n = iota + ch * L
        row = 12 * lax.shift_right_logical(n, 1) + trow + (n & 1)
        tidx_v[pl.ds(ch * L, L)] = jnp.minimum(row, 383)
    cp_t = pltpu.async_copy(tgt_hbm.at[tidx_v], t_v, sem_t)
    cp_r.wait()
    rch = []
    for ch in range(RPW // L):
        r = ridx_v[pl.ds(ch * L, L)]
        rch.append(r)
        midx_v[pl.ds(ch * L, L)] = lax.shift_right_logical(r, 7)
    cps_g = [pltpu.async_copy(ev2_hbm.at[rch[ii // L][ii % L]], g_v.at[ii],
                              sem_g)
             for ii in range(RPW)]
    cp_m = pltpu.async_copy(mean_hbm.at[midx_v], mean_v, sem_m)
    cp_x.wait()
    for cp in cps_g:
        cp.wait()
    cp_m.wait()
    cp_t.wait()

    row3 = [iota * 3 + j for j in range(3)]
    rj = [plsc.load_gather(ridx_v, [row3[j]]) for j in range(3)]
    mt = [plsc.load_gather(mean_v, [row3[j], rj[j] & 127]) for j in range(3)]
    toff = (base & 127) + iota * 3
    trow2 = lax.shift_right_logical(toff, 7)
    tcol = toff & 127

    for bc in range(B // 8):
        acc = [[mt[j] for _ in range(8)] for j in range(3)]
        for k in range(30):
            kf = jnp.full((L,), k, jnp.int32)
            col = [plsc.load_gather(g_v, [row3[j], kf]) for j in range(3)]
            for b8 in range(8):
                b = bc * 8 + b8
                xv = plsc.load_gather(
                    x_v, [jnp.full((L,), 8 + b * KX + k, jnp.int32)])
                for j in range(3):
                    acc[j][b8] = acc[j][b8] + col[j] * xv
        for b8 in range(8):
            b = bc * 8 + b8
            tt = plsc.load_gather(t_v, [trow2 + 2 * b, tcol])
            d0 = acc[0][b8] - tt
            d1 = acc[1][b8] - tt
            d2 = acc[2][b8] - tt
            dist = _nsqrt(d0 * d0 + d1 * d1 + d2 * d2)
            part_v[b, :] = jnp.broadcast_to(jnp.sum(dist), (L,))

    pltpu.sync_copy(part_v, stage.at[c, s])
    plsc.subcore_barrier()

    @pl.when(s == 0)
    def _fold():
        cps = [pltpu.async_copy(stage.at[c, i], red_v.at[i], sem_g)
               for i in range(16)]
        for cp in cps:
            cp.wait()
        for b in range(B):
            acc_r = red_v[0, b, :]
            for i in range(1, 16):
                acc_r = acc_r + red_v[i, b, :]
            osum_v[b, :] = acc_r
        pltpu.sync_copy(osum_v, out_hbm.at[c])


def kernel(output, target, eigenVectors, mean, indices, outline, rightLineIdxs):
    tgt2 = target.reshape(-1, 128)
    mean2 = mean.reshape(-1, 128)
    part = _sc_loss(eigenVectors, rightLineIdxs, output.reshape(-1), tgt2,
                    mean2)
    return (part[0] + part[1])[:, 0]


# speedup vs baseline: 1.5630x; 1.1046x over previous
import functools

import jax
import jax.numpy as jnp
from jax import lax
from jax.experimental import pallas as pl
from jax.experimental.pallas import tpu as pltpu
from jax.experimental.pallas import tpu_sc as plsc

B = 32
KX = 30
NTRI = 512
NW = 32
TPW = NTRI // NW
RPW = 3 * TPW
L = 16

_mesh = plsc.VectorSubcoreMesh(core_axis_name="c", subcore_axis_name="s")

D = 49152
BAND = D // 4


def _nsqrt(x):
    i = lax.bitcast_convert_type(x, jnp.int32)
    y = lax.bitcast_convert_type(
        lax.shift_right_arithmetic(i, 1) + jnp.int32(0x1FBD1DF5), jnp.float32)
    for _ in range(3):
        y = 0.5 * (y + x / y)
    return y


@functools.partial(
    pl.kernel,
    out_type=jax.ShapeDtypeStruct((2, B, L), jnp.float32),
    mesh=_mesh,
    compiler_params=pltpu.CompilerParams(needs_layout_passes=False,
                                         use_tc_tiling_on_sc=True),
    scratch_types=[
        pltpu.VMEM((RPW,), jnp.int32),
        pltpu.VMEM((RPW,), jnp.int32),
        pltpu.VMEM((RPW, 64), jnp.float32),
        pltpu.VMEM((RPW, 128), jnp.float32),
        pltpu.VMEM((8 + B * KX, ), jnp.float32),
        pltpu.VMEM((2 * B,), jnp.int32),
        pltpu.VMEM((2 * B, 128), jnp.float32),
        pltpu.VMEM((8, L), jnp.float32),
        pltpu.VMEM((16, B, L), jnp.float32),
        pltpu.VMEM((B, L), jnp.float32),
        pltpu.HBM((2, 16, B, L), jnp.float32),
        pltpu.SemaphoreType.DMA,
        pltpu.SemaphoreType.DMA,
        pltpu.SemaphoreType.DMA,
        pltpu.SemaphoreType.DMA,
    ],
)
def _sc_loss(ev2_hbm, rlt_hbm, x_hbm, tgt_hbm, mean_hbm, out_hbm,
             ridx_v, midx_v, g_v, mean_v, x_v, tidx_v, t_v, part_v,
             red_v, osum_v, stage, sem_g, sem_m, sem_t, sem_x):
    c = lax.axis_index("c")
    s = lax.axis_index("s")
    g = c * 16 + s
    base = g * RPW

    iota = lax.iota(jnp.int32, L)

    cp_r = pltpu.async_copy(rlt_hbm.at[pl.ds(base, RPW)], ridx_v, sem_m)
    cp_x = pltpu.async_copy(x_hbm, x_v.at[pl.ds(8, B * KX)], sem_x)
    trow = lax.shift_right_logical(base, 7)
    for ch in range(2 * B // L):
        n = iota + ch * L
        row = 12 * lax.shift_right_logical(n, 1) + trow + (n & 1)
        tidx_v[pl.ds(ch * L, L)] = jnp.minimum(row, 383)
    cp_t = pltpu.async_copy(tgt_hbm.at[tidx_v], t_v, sem_t)
    cp_r.wait()
    rch = []
    for ch in range(RPW // L):
        r = ridx_v[pl.ds(ch * L, L)]
        rch.append(r)
        midx_v[pl.ds(ch * L, L)] = lax.shift_right_logical(r, 7)
    cps_g = [pltpu.async_copy(ev2_hbm.at[rch[ii // L][ii % L]], g_v.at[ii],
                              sem_g)
             for ii in range(RPW)]
    cp_m = pltpu.async_copy(mean_hbm.at[midx_v], mean_v, sem_m)
    cp_x.wait()
    for cp in cps_g:
        cp.wait()
    cp_m.wait()
    cp_t.wait()

    row3 = [iota * 3 + j for j in range(3)]
    rj = [plsc.load_gather(ridx_v, [row3[j]]) for j in range(3)]
    mt = [plsc.load_gather(mean_v, [row3[j], rj[j] & 127]) for j in range(3)]
    toff = (base & 127) + iota * 3
    trow2 = lax.shift_right_logical(toff, 7)
    tcol = toff & 127

    def _chunk(bc, carry):
        acc = [[mt[j] for _ in range(8)] for j in range(3)]
        for k in range(30):
            kf = jnp.full((L,), k, jnp.int32)
            col = [plsc.load_gather(g_v, [row3[j], kf]) for j in range(3)]
            for b8 in range(8):
                b = bc * 8 + b8
                xv = plsc.load_gather(x_v, [iota * 0 + (8 + KX * b + k)])
                for j in range(3):
                    acc[j][b8] = acc[j][b8] + col[j] * xv
        for b8 in range(8):
            b = bc * 8 + b8
            tt = plsc.load_gather(t_v, [trow2 + 2 * b, tcol])
            d0 = acc[0][b8] - tt
            d1 = acc[1][b8] - tt
            d2 = acc[2][b8] - tt
            dist = _nsqrt(d0 * d0 + d1 * d1 + d2 * d2)
            part_v[b8, :] = jnp.broadcast_to(jnp.sum(dist), (L,))
        pltpu.sync_copy(part_v, stage.at[c, s, pl.ds(bc * 8, 8)])
        return carry

    lax.fori_loop(0, B // 8, _chunk, 0)

    plsc.subcore_barrier()

    @pl.when(s == 0)
    def _fold():
        cps = [pltpu.async_copy(stage.at[c, i], red_v.at[i], sem_g)
               for i in range(16)]
        for cp in cps:
            cp.wait()
        for b in range(B):
            acc_r = red_v[0, b, :]
            for i in range(1, 16):
                acc_r = acc_r + red_v[i, b, :]
            osum_v[b, :] = acc_r
        pltpu.sync_copy(osum_v, out_hbm.at[c])


def kernel(output, target, eigenVectors, mean, indices, outline, rightLineIdxs):
    tgt2 = target.reshape(-1, 128)
    mean2 = mean.reshape(-1, 128)
    part = _sc_loss(eigenVectors, rightLineIdxs, output.reshape(-1), tgt2,
                    mean2)
    return (part[0] + part[1])[:, 0]


# speedup vs baseline: 1.6893x; 1.0808x over previous
import functools

import jax
import jax.numpy as jnp
from jax import lax
from jax.experimental import pallas as pl
from jax.experimental.pallas import tpu as pltpu
from jax.experimental.pallas import tpu_sc as plsc

B = 32
KX = 30
NTRI = 512
NW = 32
TPW = NTRI // NW
RPW = 3 * TPW
L = 16

_mesh = plsc.VectorSubcoreMesh(core_axis_name="c", subcore_axis_name="s")

D = 49152
BAND = D // 4


def _nsqrt(x):
    i = lax.bitcast_convert_type(x, jnp.int32)
    y = lax.bitcast_convert_type(
        lax.shift_right_arithmetic(i, 1) + jnp.int32(0x1FBD1DF5), jnp.float32)
    for _ in range(3):
        y = 0.5 * (y + x / y)
    return y


@functools.partial(
    pl.kernel,
    out_type=jax.ShapeDtypeStruct((2, B, L), jnp.float32),
    mesh=_mesh,
    compiler_params=pltpu.CompilerParams(needs_layout_passes=False,
                                         use_tc_tiling_on_sc=True),
    scratch_types=[
        pltpu.VMEM((RPW,), jnp.int32),
        pltpu.VMEM((RPW,), jnp.int32),
        pltpu.VMEM((RPW, 64), jnp.float32),
        pltpu.VMEM((RPW, 128), jnp.float32),
        pltpu.VMEM((8 + B * KX, ), jnp.float32),
        pltpu.VMEM((2 * B,), jnp.int32),
        pltpu.VMEM((2 * B, 128), jnp.float32),
        pltpu.VMEM((8, L), jnp.float32),
        pltpu.VMEM((16, 2, L), jnp.float32),
        pltpu.VMEM((2, L), jnp.float32),
        pltpu.HBM((2, 16, B, L), jnp.float32),
        pltpu.SemaphoreType.DMA,
        pltpu.SemaphoreType.DMA,
        pltpu.SemaphoreType.DMA,
        pltpu.SemaphoreType.DMA,
    ],
)
def _sc_loss(ev2_hbm, rlt_hbm, x_hbm, tgt_hbm, mean_hbm, out_hbm,
             ridx_v, midx_v, g_v, mean_v, x_v, tidx_v, t_v, part_v,
             red_v, osum_v, stage, sem_g, sem_m, sem_t, sem_x):
    c = lax.axis_index("c")
    s = lax.axis_index("s")
    g = c * 16 + s
    base = g * RPW

    iota = lax.iota(jnp.int32, L)

    cp_r = pltpu.async_copy(rlt_hbm.at[pl.ds(base, RPW)], ridx_v, sem_m)
    cp_x = pltpu.async_copy(x_hbm, x_v.at[pl.ds(8, B * KX)], sem_x)
    trow = lax.shift_right_logical(base, 7)
    for ch in range(2 * B // L):
        n = iota + ch * L
        row = 12 * lax.shift_right_logical(n, 1) + trow + (n & 1)
        tidx_v[pl.ds(ch * L, L)] = jnp.minimum(row, 383)
    cp_t = pltpu.async_copy(tgt_hbm.at[tidx_v], t_v, sem_t)
    cp_r.wait()
    rch = []
    for ch in range(RPW // L):
        r = ridx_v[pl.ds(ch * L, L)]
        rch.append(r)
        midx_v[pl.ds(ch * L, L)] = lax.shift_right_logical(r, 7)
    cps_g = [pltpu.async_copy(ev2_hbm.at[rch[ii // L][ii % L]], g_v.at[ii],
                              sem_g)
             for ii in range(RPW)]
    cp_m = pltpu.async_copy(mean_hbm.at[midx_v], mean_v, sem_m)
    cp_x.wait()
    for cp in cps_g:
        cp.wait()
    cp_m.wait()
    cp_t.wait()

    row3 = [iota * 3 + j for j in range(3)]
    rj = [plsc.load_gather(ridx_v, [row3[j]]) for j in range(3)]
    mt = [plsc.load_gather(mean_v, [row3[j], rj[j] & 127]) for j in range(3)]
    toff = (base & 127) + iota * 3
    trow2 = lax.shift_right_logical(toff, 7)
    tcol = toff & 127

    def _chunk(bc, carry):
        acc = [[mt[j] for _ in range(8)] for j in range(3)]
        for k in range(30):
            kf = jnp.full((L,), k, jnp.int32)
            col = [plsc.load_gather(g_v, [row3[j], kf]) for j in range(3)]
            for b8 in range(8):
                b = bc * 8 + b8
                xv = plsc.load_gather(x_v, [iota * 0 + (8 + KX * b + k)])
                for j in range(3):
                    acc[j][b8] = acc[j][b8] + col[j] * xv
        for b8 in range(8):
            b = bc * 8 + b8
            tt = plsc.load_gather(t_v, [trow2 + 2 * b, tcol])
            d0 = acc[0][b8] - tt
            d1 = acc[1][b8] - tt
            d2 = acc[2][b8] - tt
            dist = _nsqrt(d0 * d0 + d1 * d1 + d2 * d2)
            part_v[b8, :] = jnp.broadcast_to(jnp.sum(dist), (L,))
        pltpu.sync_copy(part_v, stage.at[c, s, pl.ds(bc * 8, 8)])
        return carry

    lax.fori_loop(0, B // 8, _chunk, 0)

    plsc.subcore_barrier()

    b0 = s * 2
    cps = [pltpu.async_copy(stage.at[c, i, pl.ds(b0, 2)], red_v.at[i], sem_g)
           for i in range(16)]
    for cp in cps:
        cp.wait()
    for p in range(2):
        acc_r = red_v[0, p, :]
        for i in range(1, 16):
            acc_r = acc_r + red_v[i, p, :]
        osum_v[p, :] = acc_r
    pltpu.sync_copy(osum_v, out_hbm.at[c, pl.ds(b0, 2)])


def kernel(output, target, eigenVectors, mean, indices, outline, rightLineIdxs):
    tgt2 = target.reshape(-1, 128)
    mean2 = mean.reshape(-1, 128)
    part = _sc_loss(eigenVectors, rightLineIdxs, output.reshape(-1), tgt2,
                    mean2)
    return (part[0] + part[1])[:, 0]
